# baseline (speedup 1.0000x reference)
import functools

import jax
import jax.numpy as jnp
from jax.experimental import pallas as pl
from jax.experimental.pallas import tpu as pltpu


def _pick_tile(n, candidates):
    for c in candidates:
        if n % c == 0:
            return c
    return n


def _make_block(nsub, CK):
    def _gate(x, wsp_ref, bsp_ref):
        logits = jnp.sum(x * wsp_ref[...], axis=1, keepdims=True) + bsp_ref[0, 0]
        return jax.nn.sigmoid(logits)

    def _block(x_ref, lab_ref, wsp_ref, bsp_ref, lng_ref, lnb_ref,
               w1_ref, b1_ref, w2_ref, b2_ref, o_ref, xn_ref):
        j = pl.program_id(1)
        nj = pl.num_programs(1)

        @pl.when(j == 0)
        def _prologue():
            x = x_ref[...]
            gate = _gate(x, wsp_ref, bsp_ref)
            hs2 = x * gate
            mu = jnp.mean(hs2, axis=1, keepdims=True)
            var = jnp.mean(jnp.square(hs2 - mu), axis=1, keepdims=True)
            xn = (hs2 - mu) * jax.lax.rsqrt(var + 1e-5) * lng_ref[...] + lnb_ref[...]
            xn_ref[...] = xn.astype(jnp.bfloat16)
            o_ref[...] = hs2 + b2_ref[...]

        xn = xn_ref[...]
        for k in range(nsub):
            lo = k * CK
            h = jnp.dot(xn, w1_ref[:, lo:lo + CK],
                        preferred_element_type=jnp.float32) + b1_ref[:, lo:lo + CK]
            act = jax.nn.gelu(h).astype(jnp.bfloat16)
            o_ref[...] += jnp.dot(act, w2_ref[lo:lo + CK, :],
                                  preferred_element_type=jnp.float32)

        @pl.when(j == nj - 1)
        def _epilogue():
            gate = _gate(x_ref[...], wsp_ref, bsp_ref)
            keep = (gate >= 0.5) | (lab_ref[...] == -100)
            o_ref[...] = o_ref[...] * keep.astype(jnp.float32)

    return _block


@functools.partial(jax.jit, static_argnames=())
def _run(x, labels, W_sp, b_sp, ln_g, ln_b, W1, b1, W2, b2):
    n, h = x.shape
    dff = W1.shape[1]
    T = _pick_tile(n, (1024, 512, 256, 128, 64, 32, 16, 8))
    DK = _pick_tile(dff, (512, 256, 128))
    CK = min(DK, 256)
    nsub = DK // CK
    grid = (n // T, dff // DK)

    out = pl.pallas_call(
        _make_block(nsub, CK),
        grid=grid,
        in_specs=[
            pl.BlockSpec((T, h), lambda i, j: (i, 0)),
            pl.BlockSpec((T, 1), lambda i, j: (i, 0)),
            pl.BlockSpec((1, h), lambda i, j: (0, 0)),
            pl.BlockSpec((1, 1), lambda i, j: (0, 0)),
            pl.BlockSpec((1, h), lambda i, j: (0, 0)),
            pl.BlockSpec((1, h), lambda i, j: (0, 0)),
            pl.BlockSpec((h, DK), lambda i, j: (0, j)),
            pl.BlockSpec((1, DK), lambda i, j: (0, j)),
            pl.BlockSpec((DK, h), lambda i, j: (j, 0)),
            pl.BlockSpec((1, h), lambda i, j: (0, 0)),
        ],
        out_specs=pl.BlockSpec((T, h), lambda i, j: (i, 0)),
        out_shape=jax.ShapeDtypeStruct((n, h), jnp.float32),
        scratch_shapes=[
            pltpu.VMEM((T, h), jnp.bfloat16),
        ],
        compiler_params=pltpu.CompilerParams(
            dimension_semantics=("parallel", "arbitrary"),
        ),
    )(x, labels, W_sp, b_sp, ln_g, ln_b, W1, b1, W2, b2)
    return out


def kernel(hidden_states, labels, cos, sin, cu_seq_lens_q,
           W_sp, b_sp, ln_g, ln_b, W1, b1, W2, b2):
    b, s, h = hidden_states.shape
    dff = W1.shape[1]
    x = hidden_states.astype(jnp.float32).reshape(b * s, h)
    lab = labels.reshape(b * s, 1)
    out = _run(
        x, lab,
        W_sp.astype(jnp.float32).reshape(1, h),
        b_sp.astype(jnp.float32).reshape(1, 1),
        ln_g.astype(jnp.float32).reshape(1, h),
        ln_b.astype(jnp.float32).reshape(1, h),
        W1.astype(jnp.bfloat16),
        b1.astype(jnp.float32).reshape(1, dff),
        W2.astype(jnp.bfloat16),
        b2.astype(jnp.float32).reshape(1, h),
    )
    return out.reshape(b, s, h).astype(hidden_states.dtype)


# speedup vs baseline: 1.2507x; 1.2507x over previous
import functools

import jax
import jax.numpy as jnp
from jax.experimental import pallas as pl
from jax.experimental.pallas import tpu as pltpu


def _pick_tile(n, candidates):
    for c in candidates:
        if n % c == 0:
            return c
    return n


def _make_block(nsub, CK):
    def _gate(x, wsp_ref, bsp_ref):
        logits = jnp.sum(x * wsp_ref[...], axis=1, keepdims=True) + bsp_ref[0, 0]
        return jax.nn.sigmoid(logits)

    def _block(x_ref, lab_ref, wsp_ref, bsp_ref, lng_ref, lnb_ref,
               w1_ref, b1_ref, w2_ref, b2_ref, o_ref, xn_ref, act_ref):
        j = pl.program_id(1)
        nj = pl.num_programs(1)

        @pl.when(j == 0)
        def _prologue():
            x = x_ref[...]
            gate = _gate(x, wsp_ref, bsp_ref)
            hs2 = x * gate
            mu = jnp.mean(hs2, axis=1, keepdims=True)
            var = jnp.mean(jnp.square(hs2 - mu), axis=1, keepdims=True)
            xn = (hs2 - mu) * jax.lax.rsqrt(var + 1e-5) * lng_ref[...] + lnb_ref[...]
            xn_ref[...] = xn.astype(jnp.bfloat16)
            o_ref[...] = hs2 + b2_ref[...]

        xn = xn_ref[...]
        for k in range(nsub):
            lo = k * CK
            h = jnp.dot(xn, w1_ref[:, lo:lo + CK],
                        preferred_element_type=jnp.float32) + b1_ref[:, lo:lo + CK]
            act_ref[:, lo:lo + CK] = jax.nn.gelu(h).astype(jnp.bfloat16)
        o_ref[...] += jnp.dot(act_ref[...], w2_ref[...],
                              preferred_element_type=jnp.float32)

        @pl.when(j == nj - 1)
        def _epilogue():
            gate = _gate(x_ref[...], wsp_ref, bsp_ref)
            keep = (gate >= 0.5) | (lab_ref[...] == -100)
            o_ref[...] = o_ref[...] * keep.astype(jnp.float32)

    return _block


@functools.partial(jax.jit, static_argnames=())
def _run(x, labels, W_sp, b_sp, ln_g, ln_b, W1, b1, W2, b2):
    n, h = x.shape
    dff = W1.shape[1]
    T = _pick_tile(n, (512, 256, 128, 64, 32, 16, 8))
    DK = _pick_tile(dff, (1024, 512, 256, 128))
    CK = min(DK, 512)
    nsub = DK // CK
    grid = (n // T, dff // DK)

    out = pl.pallas_call(
        _make_block(nsub, CK),
        grid=grid,
        in_specs=[
            pl.BlockSpec((T, h), lambda i, j: (i, 0)),
            pl.BlockSpec((T, 1), lambda i, j: (i, 0)),
            pl.BlockSpec((1, h), lambda i, j: (0, 0)),
            pl.BlockSpec((1, 1), lambda i, j: (0, 0)),
            pl.BlockSpec((1, h), lambda i, j: (0, 0)),
            pl.BlockSpec((1, h), lambda i, j: (0, 0)),
            pl.BlockSpec((h, DK), lambda i, j: (0, j)),
            pl.BlockSpec((1, DK), lambda i, j: (0, j)),
            pl.BlockSpec((DK, h), lambda i, j: (j, 0)),
            pl.BlockSpec((1, h), lambda i, j: (0, 0)),
        ],
        out_specs=pl.BlockSpec((T, h), lambda i, j: (i, 0)),
        out_shape=jax.ShapeDtypeStruct((n, h), jnp.float32),
        scratch_shapes=[
            pltpu.VMEM((T, h), jnp.bfloat16),
            pltpu.VMEM((T, DK), jnp.bfloat16),
        ],
        compiler_params=pltpu.CompilerParams(
            dimension_semantics=("parallel", "arbitrary"),
        ),
    )(x, labels, W_sp, b_sp, ln_g, ln_b, W1, b1, W2, b2)
    return out


def kernel(hidden_states, labels, cos, sin, cu_seq_lens_q,
           W_sp, b_sp, ln_g, ln_b, W1, b1, W2, b2):
    b, s, h = hidden_states.shape
    dff = W1.shape[1]
    x = hidden_states.astype(jnp.float32).reshape(b * s, h)
    lab = labels.reshape(b * s, 1)
    out = _run(
        x, lab,
        W_sp.astype(jnp.float32).reshape(1, h),
        b_sp.astype(jnp.float32).reshape(1, 1),
        ln_g.astype(jnp.float32).reshape(1, h),
        ln_b.astype(jnp.float32).reshape(1, h),
        W1.astype(jnp.bfloat16),
        b1.astype(jnp.float32).reshape(1, dff),
        W2.astype(jnp.bfloat16),
        b2.astype(jnp.float32).reshape(1, h),
    )
    return out.reshape(b, s, h).astype(hidden_states.dtype)


# speedup vs baseline: 1.2775x; 1.0214x over previous
import functools

import jax
import jax.numpy as jnp
from jax.experimental import pallas as pl
from jax.experimental.pallas import tpu as pltpu


def _pick_tile(n, candidates):
    for c in candidates:
        if n % c == 0:
            return c
    return n


def _make_block(nsub, CK):
    def _gate(x, wsp_ref, bsp_ref):
        logits = jnp.sum(x * wsp_ref[...], axis=1, keepdims=True) + bsp_ref[0, 0]
        return jax.nn.sigmoid(logits)

    def _block(x_ref, lab_ref, wsp_ref, bsp_ref, lng_ref, lnb_ref,
               w1_ref, b1_ref, w2_ref, b2_ref, o_ref, xn_ref):
        j = pl.program_id(1)
        nj = pl.num_programs(1)

        @pl.when(j == 0)
        def _prologue():
            x = x_ref[...].astype(jnp.float32)
            gate = _gate(x, wsp_ref, bsp_ref)
            hs2 = x * gate
            mu = jnp.mean(hs2, axis=1, keepdims=True)
            var = jnp.mean(jnp.square(hs2 - mu), axis=1, keepdims=True)
            xn = (hs2 - mu) * jax.lax.rsqrt(var + 1e-5) * lng_ref[...] + lnb_ref[...]
            xn_ref[...] = xn.astype(jnp.bfloat16)
            o_ref[...] = hs2 + b2_ref[...]

        xn = xn_ref[...]
        for k in range(nsub):
            lo = k * CK
            h = jnp.dot(xn, w1_ref[:, lo:lo + CK],
                        preferred_element_type=jnp.float32) + b1_ref[:, lo:lo + CK]
            act = jax.nn.gelu(h).astype(jnp.bfloat16)
            o_ref[...] += jnp.dot(act, w2_ref[lo:lo + CK, :],
                                  preferred_element_type=jnp.float32)

        @pl.when(j == nj - 1)
        def _epilogue():
            gate = _gate(x_ref[...].astype(jnp.float32), wsp_ref, bsp_ref)
            klab = (lab_ref[0] == -100).astype(jnp.float32).T
            keep = jnp.maximum((gate >= 0.5).astype(jnp.float32), klab)
            o_ref[...] = o_ref[...] * keep

    return _block


@functools.partial(jax.jit, static_argnames=())
def _run(x, labels, W_sp, b_sp, ln_g, ln_b, W1, b1, W2, b2):
    n, h = x.shape
    dff = W1.shape[1]
    T = _pick_tile(n, (1024, 512, 256, 128, 64, 32, 16, 8))
    lab3 = labels.reshape(n // T, 1, T)
    DK = _pick_tile(dff, (1024, 512, 256, 128))
    CK = min(DK, 512)
    nsub = DK // CK
    grid = (n // T, dff // DK)

    out = pl.pallas_call(
        _make_block(nsub, CK),
        grid=grid,
        in_specs=[
            pl.BlockSpec((T, h), lambda i, j: (i, 0)),
            pl.BlockSpec((1, 1, T), lambda i, j: (i, 0, 0)),
            pl.BlockSpec((1, h), lambda i, j: (0, 0)),
            pl.BlockSpec((1, 1), lambda i, j: (0, 0)),
            pl.BlockSpec((1, h), lambda i, j: (0, 0)),
            pl.BlockSpec((1, h), lambda i, j: (0, 0)),
            pl.BlockSpec((h, DK), lambda i, j: (0, j)),
            pl.BlockSpec((1, DK), lambda i, j: (0, j)),
            pl.BlockSpec((DK, h), lambda i, j: (j, 0)),
            pl.BlockSpec((1, h), lambda i, j: (0, 0)),
        ],
        out_specs=pl.BlockSpec((T, h), lambda i, j: (i, 0)),
        out_shape=jax.ShapeDtypeStruct((n, h), jnp.float32),
        scratch_shapes=[
            pltpu.VMEM((T, h), jnp.bfloat16),
        ],
        compiler_params=pltpu.CompilerParams(
            dimension_semantics=("parallel", "arbitrary"),
        ),
    )(x, lab3, W_sp, b_sp, ln_g, ln_b, W1, b1, W2, b2)
    return out


def kernel(hidden_states, labels, cos, sin, cu_seq_lens_q,
           W_sp, b_sp, ln_g, ln_b, W1, b1, W2, b2):
    b, s, h = hidden_states.shape
    dff = W1.shape[1]
    x = hidden_states.astype(jnp.bfloat16).reshape(b * s, h)
    lab = labels.reshape(b * s)
    out = _run(
        x, lab,
        W_sp.astype(jnp.float32).reshape(1, h),
        b_sp.astype(jnp.float32).reshape(1, 1),
        ln_g.astype(jnp.float32).reshape(1, h),
        ln_b.astype(jnp.float32).reshape(1, h),
        W1.astype(jnp.bfloat16),
        b1.astype(jnp.float32).reshape(1, dff),
        W2.astype(jnp.bfloat16),
        b2.astype(jnp.float32).reshape(1, h),
    )
    return out.reshape(b, s, h).astype(hidden_states.dtype)


# speedup vs baseline: 1.3239x; 1.0364x over previous
import functools

import jax
import jax.numpy as jnp
from jax.experimental import pallas as pl
from jax.experimental.pallas import tpu as pltpu


def _pick_tile(n, candidates):
    for c in candidates:
        if n % c == 0:
            return c
    return n


def _make_block(nsub, CK):
    def _gate(x, wsp_ref, bsp_ref):
        logits = jnp.sum(x * wsp_ref[...], axis=1, keepdims=True) + bsp_ref[0, 0]
        return jax.nn.sigmoid(logits)

    def _block(x_ref, lab_ref, wsp_ref, bsp_ref, lng_ref, lnb_ref,
               w1_ref, b1_ref, w2_ref, b2_ref, o_ref, xn_ref):
        j = pl.program_id(1)
        nj = pl.num_programs(1)

        @pl.when(j == 0)
        def _prologue():
            x = x_ref[...]
            gate = _gate(x, wsp_ref, bsp_ref)
            hs2 = x * gate
            mu = jnp.mean(hs2, axis=1, keepdims=True)
            var = jnp.mean(jnp.square(hs2 - mu), axis=1, keepdims=True)
            xn = (hs2 - mu) * jax.lax.rsqrt(var + 1e-5) * lng_ref[...] + lnb_ref[...]
            xn_ref[...] = xn.astype(jnp.bfloat16)
            o_ref[...] = hs2 + b2_ref[...]

        xn = xn_ref[...]
        for k in range(nsub):
            lo = k * CK
            h = jnp.dot(xn, w1_ref[:, lo:lo + CK],
                        preferred_element_type=jnp.float32) + b1_ref[:, lo:lo + CK]
            act = jax.nn.gelu(h).astype(jnp.bfloat16)
            o_ref[...] += jnp.dot(act, w2_ref[lo:lo + CK, :],
                                  preferred_element_type=jnp.float32)

        @pl.when(j == nj - 1)
        def _epilogue():
            gate = _gate(x_ref[...], wsp_ref, bsp_ref)
            keep = (gate >= 0.5) | (lab_ref[...] == -100)
            o_ref[...] = o_ref[...] * keep.astype(jnp.float32)

    return _block


@functools.partial(jax.jit, static_argnames=())
def _run(x, labels, W_sp, b_sp, ln_g, ln_b, W1, b1, W2, b2):
    n, h = x.shape
    dff = W1.shape[1]
    T = _pick_tile(n, (512, 256, 128, 64, 32, 16, 8))
    DK = _pick_tile(dff, (2048, 1024, 512, 256, 128))
    CK = min(DK, 512)
    nsub = DK // CK
    grid = (n // T, dff // DK)

    out = pl.pallas_call(
        _make_block(nsub, CK),
        grid=grid,
        in_specs=[
            pl.BlockSpec((T, h), lambda i, j: (i, 0)),
            pl.BlockSpec((T, 1), lambda i, j: (i, 0)),
            pl.BlockSpec((1, h), lambda i, j: (0, 0)),
            pl.BlockSpec((1, 1), lambda i, j: (0, 0)),
            pl.BlockSpec((1, h), lambda i, j: (0, 0)),
            pl.BlockSpec((1, h), lambda i, j: (0, 0)),
            pl.BlockSpec((h, DK), lambda i, j: (0, j)),
            pl.BlockSpec((1, DK), lambda i, j: (0, j)),
            pl.BlockSpec((DK, h), lambda i, j: (j, 0)),
            pl.BlockSpec((1, h), lambda i, j: (0, 0)),
        ],
        out_specs=pl.BlockSpec((T, h), lambda i, j: (i, 0)),
        out_shape=jax.ShapeDtypeStruct((n, h), jnp.float32),
        scratch_shapes=[
            pltpu.VMEM((T, h), jnp.bfloat16),
        ],
        compiler_params=pltpu.CompilerParams(
            dimension_semantics=("parallel", "arbitrary"),
        ),
    )(x, labels, W_sp, b_sp, ln_g, ln_b, W1, b1, W2, b2)
    return out


def kernel(hidden_states, labels, cos, sin, cu_seq_lens_q,
           W_sp, b_sp, ln_g, ln_b, W1, b1, W2, b2):
    b, s, h = hidden_states.shape
    dff = W1.shape[1]
    x = hidden_states.astype(jnp.float32).reshape(b * s, h)
    lab = labels.reshape(b * s, 1)
    out = _run(
        x, lab,
        W_sp.astype(jnp.float32).reshape(1, h),
        b_sp.astype(jnp.float32).reshape(1, 1),
        ln_g.astype(jnp.float32).reshape(1, h),
        ln_b.astype(jnp.float32).reshape(1, h),
        W1.astype(jnp.bfloat16),
        b1.astype(jnp.float32).reshape(1, dff),
        W2.astype(jnp.bfloat16),
        b2.astype(jnp.float32).reshape(1, h),
    )
    return out.reshape(b, s, h).astype(hidden_states.dtype)
